# baseline (speedup 1.0000x reference)
import jax
import jax.numpy as jnp
from jax import lax
from jax.experimental import pallas as pl
from jax.experimental.pallas import tpu as pltpu
from jax.experimental.pallas import tpu_sc as plsc

F = 26
V = 100000
D = 32
B = 16384
FD = F * D
QF = V // 4
NBLK = 781
VTAIL = NBLK * 128
NC, NS, L = 2, 16, 16
FPC = F // NC
P1N = FPC * NBLK
P1PT = (P1N + NS - 1) // NS
UPT = (B // 128) * FPC // NS


def _body(idx_hbm, tabn_hbm, tails_hbm, out1_hbm, out_hbm,
          tbuf, obuf, ibuf, grbuf, sbuf, gbuf, obuf2, gsem):
    c = lax.axis_index("c")
    s = lax.axis_index("s")
    iota = lax.iota(jnp.int32, L)
    zero16 = jnp.zeros((L,), jnp.int32)

    def p1(i, _):
        blk = i * NS + s

        @pl.when(blk < P1N)
        def _():
            fl = blk // NBLK
            vb = blk % NBLK
            f = c * FPC + fl
            for db in range(4):
                pltpu.sync_copy(
                    tabn_hbm.at[pl.ds((f * 4 + db) * 8, 8),
                                pl.ds(vb * 128, 128)],
                    tbuf.at[pl.ds(db * 8, 8), :])

            def trow(q, _):
                for k in range(8):
                    rows = iota + (k % 2) * L
                    cols = zero16 + (q * 4 + k // 2)
                    obuf[q, pl.ds(k * L, L)] = plsc.load_gather(
                        tbuf, [rows, cols])
                return 0

            lax.fori_loop(0, 32, trow, 0)
            pltpu.sync_copy(
                obuf, out1_hbm.at[pl.ds(f * QF + vb * 32, 32), :])
        return 0

    lax.fori_loop(0, P1PT, p1, 0)

    @pl.when(s == 0)
    def _():
        def ptail(fl, _):
            f = c * FPC + fl
            pltpu.sync_copy(tails_hbm.at[f],
                            out1_hbm.at[pl.ds(f * QF + VTAIL // 4, 8), :])
            return 0
        lax.fori_loop(0, FPC, ptail, 0)

    plsc.subcore_barrier()

    def p2(u, _):
        g = s * UPT + u
        bb = g // FPC
        f = c * FPC + (g % FPC)
        pltpu.sync_copy(
            idx_hbm.at[pl.ds((f // 8) * 8, 8), pl.ds(bb * 128, 128)], ibuf)
        row = f % 8
        for k in range(8):
            v = ibuf[row, pl.ds(k * L, L)]
            grbuf[pl.ds(k * L, L)] = f * QF + lax.shift_right_logical(v, 2)
            sbuf[pl.ds(k * L, L)] = lax.shift_left(v & 3, 5)
        pltpu.async_copy(out1_hbm.at[grbuf], gbuf, gsem).wait()

        def drow(d, _):
            for k in range(8):
                rows = iota + k * L
                cols = sbuf[pl.ds(k * L, L)] + d
                obuf2[d, pl.ds(k * L, L)] = plsc.load_gather(
                    gbuf, [rows, cols])
            return 0

        lax.fori_loop(0, D, drow, 0)
        pltpu.sync_copy(
            obuf2, out_hbm.at[pl.ds(f * D, D), pl.ds(bb * 128, 128)])
        return 0

    lax.fori_loop(0, UPT, p2, 0)


@jax.jit
def _run(idxT, tabn, tails):
    mesh = plsc.VectorSubcoreMesh(core_axis_name="c", subcore_axis_name="s")
    kfn = pl.kernel(
        _body,
        out_type=(
            jax.ShapeDtypeStruct((F * QF, 128), jnp.float32),
            jax.ShapeDtypeStruct((FD, B), jnp.float32),
        ),
        mesh=mesh,
        compiler_params=pltpu.CompilerParams(use_tc_tiling_on_sc=True,
                                             needs_layout_passes=False),
        scratch_types=[
            pltpu.VMEM((32, 128), jnp.float32),
            pltpu.VMEM((32, 128), jnp.float32),
            pltpu.VMEM((8, 128), jnp.int32),
            pltpu.VMEM((128,), jnp.int32),
            pltpu.VMEM((128,), jnp.int32),
            pltpu.VMEM((128, 128), jnp.float32),
            pltpu.VMEM((32, 128), jnp.float32),
            pltpu.SemaphoreType.DMA,
        ],
    )
    return kfn(idxT, tabn, tails)


def kernel(tensor, tables):
    idxT = jnp.pad(tensor.astype(jnp.int32).T, ((0, 32 - F), (0, 0)))
    tabn = tables.transpose(0, 2, 1).reshape(FD, V)
    tails = tables[:, VTAIL:, :].reshape(F, 8, 128)
    _, out = _run(idxT, tabn, tails)
    return out.T


# speedup vs baseline: 1.6820x; 1.6820x over previous
import jax
import jax.numpy as jnp
from jax import lax
from jax.experimental import pallas as pl
from jax.experimental.pallas import tpu as pltpu
from jax.experimental.pallas import tpu_sc as plsc

F = 26
V = 100000
D = 32
B = 16384
FD = F * D
QF = V // 4
NC, NS, L = 2, 16, 16
FPC = F // NC
SBW = 512
NSB = 195
VSING = NSB * SBW
VTAIL = VSING + 128
P1N = FPC * NSB
UPT = (B // 128) * FPC // NS


def _transpose_block(src, dst, width_q, iota):
    def grp(qi, _):
        for qq in range(8):
            q = qi * 8 + qq
            for k in range(8):
                rows = iota + (k % 2) * L
                cols = (q * 4 + k // 2) + jnp.zeros((L,), jnp.int32)
                dst[q, pl.ds(k * L, L)] = plsc.load_gather(src, [rows, cols])
        return 0
    lax.fori_loop(0, width_q // 8, grp, 0)


def _body(idx_hbm, tabn_hbm, tails_hbm, out1_hbm, out_hbm,
          tbuf0, tbuf1, pobuf0, pobuf1, ibuf, grbuf0, grbuf1,
          sbuf0, sbuf1, gbuf0, gbuf1, obuf2a, obuf2b,
          lsem0, lsem1, ssem0, ssem1, gsem0, gsem1, osem0, osem1):
    c = lax.axis_index("c")
    s = lax.axis_index("s")
    iota = lax.iota(jnp.int32, L)
    pobufs = (pobuf0, pobuf1)
    tbufs = (tbuf0, tbuf1)
    lsems = (lsem0, lsem1)
    ssems = (ssem0, ssem1)

    def p1_load(g, par):
        @pl.when(g < P1N)
        def _():
            fl = g // NSB
            sb = g % NSB
            f = c * FPC + fl
            pltpu.async_copy(
                tabn_hbm.at[pl.ds(f * D, D), pl.ds(sb * SBW, SBW)],
                tbufs[par], lsems[par])

    def p1_store(g, par):
        @pl.when(g < P1N)
        def _():
            fl = g // NSB
            sb = g % NSB
            f = c * FPC + fl
            pltpu.async_copy(
                pobufs[par],
                out1_hbm.at[pl.ds(f * QF + sb * (SBW // 4), SBW // 4), :],
                ssems[par])

    def p1_wait_load(g, par):
        @pl.when(g < P1N)
        def _():
            pltpu.make_async_copy(tabn_hbm.at[pl.ds(0, D), pl.ds(0, SBW)],
                                  tbufs[par], lsems[par]).wait()

    def p1_wait_store(g, par):
        @pl.when((g >= 0) & (g < P1N))
        def _():
            pltpu.make_async_copy(
                pobufs[par],
                out1_hbm.at[pl.ds(0, SBW // 4), :], ssems[par]).wait()

    p1_load(s, 0)

    def p1_iter(i, _):
        g0 = i * 2 * NS + s
        g1 = (i * 2 + 1) * NS + s
        p1_wait_load(g0, 0)
        p1_load(g1, 1)
        p1_wait_store(g0 - 2 * NS, 0)
        _transpose_block(tbuf0, pobuf0, 128, iota)
        p1_store(g0, 0)
        p1_wait_load(g1, 1)
        p1_load(g0 + 2 * NS, 0)
        p1_wait_store(g1 - 2 * NS, 1)
        _transpose_block(tbuf1, pobuf1, 128, iota)
        p1_store(g1, 1)
        return 0

    lax.fori_loop(0, 80, p1_iter, 0)
    p1_wait_store(2 * 79 * NS + s, 0)
    p1_wait_store((2 * 79 + 1) * NS + s, 1)

    @pl.when(s < FPC)
    def _():
        f = c * FPC + s
        pltpu.sync_copy(
            tabn_hbm.at[pl.ds(f * D, D), pl.ds(VSING, 128)],
            tbuf0.at[:, pl.ds(0, 128)])
        _transpose_block(tbuf0, pobuf0, 32, iota)
        pltpu.sync_copy(
            pobuf0.at[pl.ds(0, 32), :],
            out1_hbm.at[pl.ds(f * QF + VSING // 4, 32), :])

    @pl.when(s == FPC)
    def _():
        def ptail(fl, _):
            f = c * FPC + fl
            pltpu.sync_copy(tails_hbm.at[f],
                            out1_hbm.at[pl.ds(f * QF + VTAIL // 4, 8), :])
            return 0
        lax.fori_loop(0, FPC, ptail, 0)

    plsc.subcore_barrier()

    def p2_prep(u, grbuf, sbuf):
        g = s * UPT + u
        bb = g // FPC
        f = c * FPC + (g % FPC)
        pltpu.sync_copy(
            idx_hbm.at[pl.ds((f // 8) * 8, 8), pl.ds(bb * 128, 128)], ibuf)
        row = f % 8
        fq = f * QF
        for k in range(8):
            v = ibuf[row, pl.ds(k * L, L)]
            grbuf[pl.ds(k * L, L)] = fq + lax.shift_right_logical(v, 2)
            sbuf[pl.ds(k * L, L)] = (v & 3) * 32

    def p2_select_store(u, gbuf, sbuf, obuf2, osem):
        def dgrp(di, _):
            for dd in range(4):
                d = di * 4 + dd
                for k in range(8):
                    rows = iota + k * L
                    cols = sbuf[pl.ds(k * L, L)] + d
                    obuf2[d, pl.ds(k * L, L)] = plsc.load_gather(
                        gbuf, [rows, cols])
            return 0
        lax.fori_loop(0, 8, dgrp, 0)
        g = s * UPT + u
        bb = g // FPC
        f = c * FPC + (g % FPC)
        pltpu.async_copy(
            obuf2, out_hbm.at[pl.ds(f * D, D), pl.ds(bb * 128, 128)], osem)

    def p2_wait_store(osem, obuf2):
        pltpu.make_async_copy(
            obuf2, out_hbm.at[pl.ds(0, D), pl.ds(0, 128)], osem).wait()

    p2_prep(0, grbuf0, sbuf0)
    pltpu.async_copy(out1_hbm.at[grbuf0], gbuf0, gsem0)

    def p2_iter(i, _):
        u0 = i * 2
        u1 = i * 2 + 1
        p2_prep(u1, grbuf1, sbuf1)
        pltpu.async_copy(out1_hbm.at[grbuf1], gbuf1, gsem1)

        @pl.when(i > 0)
        def _():
            p2_wait_store(osem0, obuf2a)
        pltpu.make_async_copy(out1_hbm.at[grbuf0], gbuf0, gsem0).wait()
        p2_select_store(u0, gbuf0, sbuf0, obuf2a, osem0)

        @pl.when(u1 + 1 < UPT)
        def _():
            p2_prep(u1 + 1, grbuf0, sbuf0)
            pltpu.async_copy(out1_hbm.at[grbuf0], gbuf0, gsem0)

        @pl.when(i > 0)
        def _():
            p2_wait_store(osem1, obuf2b)
        pltpu.make_async_copy(out1_hbm.at[grbuf1], gbuf1, gsem1).wait()
        p2_select_store(u1, gbuf1, sbuf1, obuf2b, osem1)
        return 0

    lax.fori_loop(0, UPT // 2, p2_iter, 0)
    p2_wait_store(osem0, obuf2a)
    p2_wait_store(osem1, obuf2b)


@jax.jit
def _run(idxT, tabn, tails):
    mesh = plsc.VectorSubcoreMesh(core_axis_name="c", subcore_axis_name="s")
    kfn = pl.kernel(
        _body,
        out_type=(
            jax.ShapeDtypeStruct((F * QF, 128), jnp.float32),
            jax.ShapeDtypeStruct((FD, B), jnp.float32),
        ),
        mesh=mesh,
        compiler_params=pltpu.CompilerParams(use_tc_tiling_on_sc=True,
                                             needs_layout_passes=False),
        scratch_types=[
            pltpu.VMEM((D, SBW), jnp.float32),
            pltpu.VMEM((D, SBW), jnp.float32),
            pltpu.VMEM((SBW // 4, 128), jnp.float32),
            pltpu.VMEM((SBW // 4, 128), jnp.float32),
            pltpu.VMEM((8, 128), jnp.int32),
            pltpu.VMEM((128,), jnp.int32),
            pltpu.VMEM((128,), jnp.int32),
            pltpu.VMEM((128,), jnp.int32),
            pltpu.VMEM((128,), jnp.int32),
            pltpu.VMEM((128, 128), jnp.float32),
            pltpu.VMEM((128, 128), jnp.float32),
            pltpu.VMEM((D, 128), jnp.float32),
            pltpu.VMEM((D, 128), jnp.float32),
            pltpu.SemaphoreType.DMA,
            pltpu.SemaphoreType.DMA,
            pltpu.SemaphoreType.DMA,
            pltpu.SemaphoreType.DMA,
            pltpu.SemaphoreType.DMA,
            pltpu.SemaphoreType.DMA,
            pltpu.SemaphoreType.DMA,
            pltpu.SemaphoreType.DMA,
        ],
    )
    return kfn(idxT, tabn, tails)


def kernel(tensor, tables):
    idxT = jnp.pad(tensor.astype(jnp.int32).T, ((0, 32 - F), (0, 0)))
    tabn = tables.transpose(0, 2, 1).reshape(FD, V)
    tails = tables[:, VTAIL:, :].reshape(F, 8, 128)
    _, out = _run(idxT, tabn, tails)
    return out.T


# speedup vs baseline: 3.2861x; 1.9537x over previous
import jax
import jax.numpy as jnp
from jax import lax
from jax.experimental import pallas as pl
from jax.experimental.pallas import tpu as pltpu
from jax.experimental.pallas import tpu_sc as plsc

F = 26
V = 100000
D = 32
B = 16384
BF = B * F
NC, NS, L = 2, 16, 16
NW = NC * NS
PER_W = BF // NW
C = 1664
NCH = PER_W // C


def _body(idx_hbm, tab_hbm, out_hbm, idx_v, buf0, buf1,
          gsem0, gsem1, ssem0, ssem1):
    wid = lax.axis_index("s") * NC + lax.axis_index("c")
    base = wid * PER_W

    pltpu.sync_copy(idx_hbm.at[pl.ds(base, PER_W)], idx_v)

    bufs = (buf0, buf1)
    gsems = (gsem0, gsem1)
    ssems = (ssem0, ssem1)

    def _gather(j):
        return pltpu.async_copy(tab_hbm.at[idx_v.at[pl.ds(j * C, C)]],
                                bufs[j % 2], gsems[j % 2])

    def _store(j):
        return pltpu.async_copy(bufs[j % 2],
                                out_hbm.at[pl.ds(base + j * C, C)],
                                ssems[j % 2])

    gathers = {0: _gather(0)}
    stores = {}
    for j in range(NCH):
        if j + 1 < NCH:
            if j >= 1:
                stores[j - 1].wait()
            gathers[j + 1] = _gather(j + 1)
        gathers[j].wait()
        stores[j] = _store(j)
    stores[NCH - 2].wait()
    stores[NCH - 1].wait()


@jax.jit
def _run(idx, flat_tables):
    mesh = plsc.VectorSubcoreMesh(core_axis_name="c", subcore_axis_name="s")
    kfn = pl.kernel(
        _body,
        out_type=jax.ShapeDtypeStruct((BF, D), jnp.float32),
        mesh=mesh,
        compiler_params=pltpu.CompilerParams(use_tc_tiling_on_sc=False),
        scratch_types=[
            pltpu.VMEM((PER_W,), jnp.int32),
            pltpu.VMEM((C, D), jnp.float32),
            pltpu.VMEM((C, D), jnp.float32),
            pltpu.SemaphoreType.DMA,
            pltpu.SemaphoreType.DMA,
            pltpu.SemaphoreType.DMA,
            pltpu.SemaphoreType.DMA,
        ],
    )
    return kfn(idx, flat_tables)


def kernel(tensor, tables):
    offs = jnp.arange(F, dtype=jnp.int32) * V
    gidx = (tensor.astype(jnp.int32) + offs[None, :]).reshape(BF)
    flat_tables = tables.reshape(F * V, D)
    out = _run(gidx, flat_tables)
    return out.reshape(B, F * D)
